# baseline (speedup 1.0000x reference)
import functools

import jax
import jax.numpy as jnp
from jax import lax
from jax.experimental import pallas as pl
from jax.experimental.pallas import tpu as pltpu
from jax.experimental.pallas import tpu_sc as plsc

_ROWS = 8192
_COLS = 1024
_NWORKERS = 32
_ROWS_PER_W = _ROWS // _NWORKERS

_mesh = plsc.VectorSubcoreMesh(core_axis_name="c", subcore_axis_name="s")


@functools.partial(
    pl.kernel,
    out_type=jax.ShapeDtypeStruct((_ROWS, _COLS), jnp.float32),
    mesh=_mesh,
)
def _sc_copy(x_hbm, o_hbm):
    wid = lax.axis_index("s") * 2 + lax.axis_index("c")
    base = wid * _ROWS_PER_W
    pltpu.sync_copy(
        x_hbm.at[pl.ds(base, _ROWS_PER_W), :],
        o_hbm.at[pl.ds(base, _ROWS_PER_W), :],
    )


def kernel(x):
    gathered = _sc_copy(x)
    sizes = jnp.array([x.shape[0]], dtype=jnp.int64)
    return (gathered, sizes)


# speedup vs baseline: 23.1322x; 23.1322x over previous
import functools

import jax
import jax.numpy as jnp
from jax import lax
from jax.experimental import pallas as pl
from jax.experimental.pallas import tpu as pltpu
from jax.experimental.pallas import tpu_sc as plsc

_ROWS = 8192
_COLS = 1024
_NWORKERS = 32
_ROWS_PER_W = _ROWS // _NWORKERS
_CHUNK_ROWS = 32
_NCHUNKS = _ROWS_PER_W // _CHUNK_ROWS
_NBUF = 2

_mesh = plsc.VectorSubcoreMesh(core_axis_name="c", subcore_axis_name="s")


@functools.partial(
    pl.kernel,
    out_type=jax.ShapeDtypeStruct((_ROWS, _COLS), jnp.float32),
    mesh=_mesh,
    scratch_types=[
        pltpu.VMEM((_NBUF, _CHUNK_ROWS, _COLS), jnp.float32),
        pltpu.SemaphoreType.DMA((_NBUF,)),
        pltpu.SemaphoreType.DMA((_NBUF,)),
    ],
)
def _sc_copy(x_hbm, o_hbm, bufs, load_sems, store_sems):
    wid = lax.axis_index("s") * 2 + lax.axis_index("c")
    base = wid * _ROWS_PER_W

    def load(i, b):
        return pltpu.make_async_copy(
            x_hbm.at[pl.ds(base + i * _CHUNK_ROWS, _CHUNK_ROWS), :],
            bufs.at[b],
            load_sems.at[b],
        )

    def store(i, b):
        return pltpu.make_async_copy(
            bufs.at[b],
            o_hbm.at[pl.ds(base + i * _CHUNK_ROWS, _CHUNK_ROWS), :],
            store_sems.at[b],
        )

    for i in range(_NBUF):
        load(i, i).start()
    for i in range(_NCHUNKS):
        b = i % _NBUF
        load(i, b).wait()
        store(i, b).start()
        nxt = i + _NBUF
        if nxt < _NCHUNKS:
            store(nxt - _NBUF, b).wait()
            load(nxt, b).start()
    for i in range(_NCHUNKS - _NBUF, _NCHUNKS):
        store(i, i % _NBUF).wait()


def kernel(x):
    gathered = _sc_copy(x)
    sizes = jnp.array([x.shape[0]], dtype=jnp.int64)
    return (gathered, sizes)
